# Initial kernel scaffold; baseline (speedup 1.0000x reference)
import jax
import jax.numpy as jnp
from jax.experimental import pallas as pl


def kernel(x, batch, gamma, log_delta, running_mean, running_var, target_batch):
    raise NotImplementedError("write your pallas kernel here")


# speedup vs baseline: 6.6599x; 6.6599x over previous
import functools

import jax
import jax.numpy as jnp
from jax import lax
from jax.experimental import pallas as pl
from jax.experimental.pallas import tpu as pltpu
from jax.experimental.pallas import tpu_sc as plsc


def _sc_gather_body(gamma_hbm, ld_hbm, idx_hbm, g_out, ld_out,
                    idx_v, g_v, ld_v, sem):
    cid = lax.axis_index("c")
    sid = lax.axis_index("s")

    @pl.when(jnp.logical_and(cid == 0, sid == 0))
    def _():
        pltpu.sync_copy(idx_hbm, idx_v)
        pltpu.async_copy(gamma_hbm.at[idx_v], g_v, sem).wait()
        pltpu.async_copy(ld_hbm.at[idx_v], ld_v, sem).wait()
        pltpu.sync_copy(g_v, g_out)
        pltpu.sync_copy(ld_v, ld_out)


def _sc_gather(gamma, log_delta, idx):
    n = idx.shape[0]
    c = gamma.shape[1]
    fn = pl.kernel(
        _sc_gather_body,
        mesh=plsc.VectorSubcoreMesh(core_axis_name="c", subcore_axis_name="s"),
        out_type=[jax.ShapeDtypeStruct((n, c), jnp.float32),
                  jax.ShapeDtypeStruct((n, c), jnp.float32)],
        scratch_types=[pltpu.VMEM((n,), jnp.int32),
                       pltpu.VMEM((n, c), jnp.float32),
                       pltpu.VMEM((n, c), jnp.float32),
                       pltpu.SemaphoreType.DMA],
    )
    return fn(gamma, log_delta, idx)


def _apply_body(x_ref, g_ref, ld_ref, gt_ref, ldt_ref, m_ref, v_ref, o_ref):
    gb = g_ref[0]
    ldb = ld_ref[0]
    gt = gt_ref[0]
    ldt = ldt_ref[0]
    mean = m_ref[0]
    var = v_ref[0]

    sv = jnp.sqrt(var + 1e-8)
    r = jnp.sqrt(jnp.exp(ldt)) / jnp.sqrt(jnp.exp(ldb) + 1e-8)
    off = mean * (1.0 - r) + sv * (gt - gb * r)
    o_ref[0] = x_ref[0] * r + off


def _apply(xr, g_sel, ld_sel, g_tgt, ld_tgt, mean3, var3, chunk):
    b, c, hw = xr.shape
    grid = (b, hw // chunk)
    par_spec = pl.BlockSpec((1, c, 1), lambda i, j: (i, 0, 0))
    fix_spec = pl.BlockSpec((1, c, 1), lambda i, j: (0, 0, 0))
    return pl.pallas_call(
        _apply_body,
        grid=grid,
        in_specs=[
            pl.BlockSpec((1, c, chunk), lambda i, j: (i, 0, j)),
            par_spec, par_spec, fix_spec, fix_spec, fix_spec, fix_spec,
        ],
        out_specs=pl.BlockSpec((1, c, chunk), lambda i, j: (i, 0, j)),
        out_shape=jax.ShapeDtypeStruct((b, c, hw), jnp.float32),
        compiler_params=pltpu.CompilerParams(
            dimension_semantics=("parallel", "parallel")),
    )(xr, g_sel, ld_sel, g_tgt, ld_tgt, mean3, var3)


def kernel(x, batch, gamma, log_delta, running_mean, running_var, target_batch):
    b, c, h, w = x.shape
    hw = h * w

    tgt = jnp.full((8,), target_batch, dtype=jnp.int32)
    idx = jnp.concatenate([batch.astype(jnp.int32), tgt])

    cp = ((c + 127) // 128) * 128
    pad = ((0, 0), (0, cp - c))
    g_all, ld_all = _sc_gather(jnp.pad(gamma, pad), jnp.pad(log_delta, pad), idx)
    g_all = g_all[:, :c]
    ld_all = ld_all[:, :c]

    g_sel = g_all[:b].reshape(b, c, 1)
    ld_sel = ld_all[:b].reshape(b, c, 1)
    g_tgt = g_all[b:b + 1].reshape(1, c, 1)
    ld_tgt = ld_all[b:b + 1].reshape(1, c, 1)
    mean3 = running_mean.reshape(1, c, 1)
    var3 = running_var.reshape(1, c, 1)

    xr = x.reshape(b, c, hw)
    out = _apply(xr, g_sel, ld_sel, g_tgt, ld_tgt, mean3, var3, chunk=6272)
    return out.reshape(b, c, h, w)


# speedup vs baseline: 21.2369x; 3.1888x over previous
import functools

import jax
import jax.numpy as jnp
from jax import lax
from jax.experimental import pallas as pl
from jax.experimental.pallas import tpu as pltpu
from jax.experimental.pallas import tpu_sc as plsc


def _sc_gather_body(gamma_hbm, ld_hbm, idx_hbm, g_out, ld_out,
                    idx_v, g_v, ld_v, sem):
    cid = lax.axis_index("c")
    sid = lax.axis_index("s")

    @pl.when(jnp.logical_and(cid == 0, sid == 0))
    def _():
        pltpu.sync_copy(idx_hbm, idx_v)
        pltpu.async_copy(gamma_hbm.at[idx_v], g_v, sem).wait()
        pltpu.async_copy(ld_hbm.at[idx_v], ld_v, sem).wait()
        pltpu.sync_copy(g_v, g_out)
        pltpu.sync_copy(ld_v, ld_out)


def _sc_gather(gamma, log_delta, idx):
    n = idx.shape[0]
    c = gamma.shape[1]
    fn = pl.kernel(
        _sc_gather_body,
        mesh=plsc.VectorSubcoreMesh(core_axis_name="c", subcore_axis_name="s"),
        out_type=[jax.ShapeDtypeStruct((n, c), jnp.float32),
                  jax.ShapeDtypeStruct((n, c), jnp.float32)],
        scratch_types=[pltpu.VMEM((n,), jnp.int32),
                       pltpu.VMEM((n, c), jnp.float32),
                       pltpu.VMEM((n, c), jnp.float32),
                       pltpu.SemaphoreType.DMA],
    )
    return fn(gamma, log_delta, idx)


def _apply_body(x_ref, g_ref, ld_ref, gt_ref, ldt_ref, m_ref, v_ref, o_ref):
    gb = g_ref[...]
    ldb = ld_ref[...]
    gt = gt_ref[...]
    ldt = ldt_ref[...]
    mean = m_ref[...]
    var = v_ref[...]

    sv = jnp.sqrt(var + 1e-8)
    r = jnp.sqrt(jnp.exp(ldt)) / jnp.sqrt(jnp.exp(ldb) + 1e-8)
    off = mean * (1.0 - r) + sv * (gt - gb * r)
    o_ref[...] = x_ref[...] * r + off


def _apply(x3, g_row, ld_row, g_tgt, ld_tgt, mean3, var3, rb):
    n, h, w = x3.shape
    c = mean3.shape[0]
    nper = c // rb
    grid = (n // rb,)
    row_spec = pl.BlockSpec((rb, 1, 1), lambda i: (i, 0, 0))
    per_spec = pl.BlockSpec((rb, 1, 1), lambda i: (i % nper, 0, 0))
    return pl.pallas_call(
        _apply_body,
        grid=grid,
        in_specs=[
            pl.BlockSpec((rb, h, w), lambda i: (i, 0, 0)),
            row_spec, row_spec, per_spec, per_spec, per_spec, per_spec,
        ],
        out_specs=pl.BlockSpec((rb, h, w), lambda i: (i, 0, 0)),
        out_shape=jax.ShapeDtypeStruct((n, h, w), jnp.float32),
        compiler_params=pltpu.CompilerParams(
            dimension_semantics=("parallel",)),
    )(x3, g_row, ld_row, g_tgt, ld_tgt, mean3, var3)


def kernel(x, batch, gamma, log_delta, running_mean, running_var, target_batch):
    b, c, h, w = x.shape
    hw = h * w

    tgt = jnp.full((8,), target_batch, dtype=jnp.int32)
    idx = jnp.concatenate([batch.astype(jnp.int32), tgt])

    cp = ((c + 127) // 128) * 128
    pad = ((0, 0), (0, cp - c))
    g_all, ld_all = _sc_gather(jnp.pad(gamma, pad), jnp.pad(log_delta, pad), idx)
    g_all = g_all[:, :c]
    ld_all = ld_all[:, :c]

    g_row = g_all[:b].reshape(b * c, 1, 1)
    ld_row = ld_all[:b].reshape(b * c, 1, 1)
    g_tgt = g_all[b].reshape(c, 1, 1)
    ld_tgt = ld_all[b].reshape(c, 1, 1)
    mean3 = running_mean.reshape(c, 1, 1)
    var3 = running_var.reshape(c, 1, 1)

    x3 = x.reshape(b * c, h, w)
    out = _apply(x3, g_row, ld_row, g_tgt, ld_tgt, mean3, var3, rb=16)
    return out.reshape(b, c, h, w)


# speedup vs baseline: 21.7557x; 1.0244x over previous
import functools

import jax
import jax.numpy as jnp
from jax import lax
from jax.experimental import pallas as pl
from jax.experimental.pallas import tpu as pltpu
from jax.experimental.pallas import tpu_sc as plsc


def _sc_gather_body(gamma_hbm, ld_hbm, idx_hbm, g_out, ld_out,
                    idx_v, g_v, ld_v, sem):
    cid = lax.axis_index("c")
    sid = lax.axis_index("s")

    @pl.when(jnp.logical_and(cid == 0, sid == 0))
    def _():
        pltpu.sync_copy(idx_hbm, idx_v)
        pltpu.async_copy(gamma_hbm.at[idx_v], g_v, sem).wait()
        pltpu.async_copy(ld_hbm.at[idx_v], ld_v, sem).wait()
        pltpu.sync_copy(g_v, g_out)
        pltpu.sync_copy(ld_v, ld_out)


def _sc_gather(gamma, log_delta, idx):
    n = idx.shape[0]
    c = gamma.shape[1]
    fn = pl.kernel(
        _sc_gather_body,
        mesh=plsc.VectorSubcoreMesh(core_axis_name="c", subcore_axis_name="s"),
        out_type=[jax.ShapeDtypeStruct((n, c), jnp.float32),
                  jax.ShapeDtypeStruct((n, c), jnp.float32)],
        scratch_types=[pltpu.VMEM((n,), jnp.int32),
                       pltpu.VMEM((n, c), jnp.float32),
                       pltpu.VMEM((n, c), jnp.float32),
                       pltpu.SemaphoreType.DMA],
    )
    return fn(gamma, log_delta, idx)


def _apply_body(x_ref, g_ref, ld_ref, gt_ref, ldt_ref, m_ref, v_ref, o_ref):
    gb = g_ref[...]
    ldb = ld_ref[...]
    gt = gt_ref[...]
    ldt = ldt_ref[...]
    mean = m_ref[...]
    var = v_ref[...]

    sv = jnp.sqrt(var + 1e-8)
    r = jnp.sqrt(jnp.exp(ldt)) / jnp.sqrt(jnp.exp(ldb) + 1e-8)
    off = mean * (1.0 - r) + sv * (gt - gb * r)
    o_ref[...] = x_ref[...] * r + off


def _apply(x3, g_row, ld_row, g_tgt, ld_tgt, mean3, var3, rb):
    n, h, w = x3.shape
    c = mean3.shape[0]
    nper = c // rb
    grid = (n // rb,)
    row_spec = pl.BlockSpec((rb, 1, 1), lambda i: (i, 0, 0))
    per_spec = pl.BlockSpec((rb, 1, 1), lambda i: (i % nper, 0, 0))
    return pl.pallas_call(
        _apply_body,
        grid=grid,
        in_specs=[
            pl.BlockSpec((rb, h, w), lambda i: (i, 0, 0)),
            row_spec, row_spec, per_spec, per_spec, per_spec, per_spec,
        ],
        out_specs=pl.BlockSpec((rb, h, w), lambda i: (i, 0, 0)),
        out_shape=jax.ShapeDtypeStruct((n, h, w), jnp.float32),
        compiler_params=pltpu.CompilerParams(
            dimension_semantics=("parallel",)),
    )(x3, g_row, ld_row, g_tgt, ld_tgt, mean3, var3)


def kernel(x, batch, gamma, log_delta, running_mean, running_var, target_batch):
    b, c, h, w = x.shape
    hw = h * w

    tgt = jnp.full((8,), target_batch, dtype=jnp.int32)
    idx = jnp.concatenate([batch.astype(jnp.int32), tgt])

    cp = ((c + 127) // 128) * 128
    pad = ((0, 0), (0, cp - c))
    g_all, ld_all = _sc_gather(jnp.pad(gamma, pad), jnp.pad(log_delta, pad), idx)
    g_all = g_all[:, :c]
    ld_all = ld_all[:, :c]

    g_row = g_all[:b].reshape(b * c, 1, 1)
    ld_row = ld_all[:b].reshape(b * c, 1, 1)
    g_tgt = g_all[b].reshape(c, 1, 1)
    ld_tgt = ld_all[b].reshape(c, 1, 1)
    mean3 = running_mean.reshape(c, 1, 1)
    var3 = running_var.reshape(c, 1, 1)

    x3 = x.reshape(b * c, h, w)
    out = _apply(x3, g_row, ld_row, g_tgt, ld_tgt, mean3, var3, rb=32)
    return out.reshape(b, c, h, w)


# speedup vs baseline: 21.8187x; 1.0029x over previous
import functools

import jax
import jax.numpy as jnp
from jax import lax
from jax.experimental import pallas as pl
from jax.experimental.pallas import tpu as pltpu
from jax.experimental.pallas import tpu_sc as plsc


def _sc_gather_body(gamma_hbm, ld_hbm, idx_hbm, g_out, ld_out,
                    idx_v, g_v, ld_v, sem):
    cid = lax.axis_index("c")
    sid = lax.axis_index("s")

    @pl.when(jnp.logical_and(cid == 0, sid == 0))
    def _():
        pltpu.sync_copy(idx_hbm, idx_v)
        pltpu.async_copy(gamma_hbm.at[idx_v], g_v, sem).wait()
        pltpu.async_copy(ld_hbm.at[idx_v], ld_v, sem).wait()
        pltpu.sync_copy(g_v, g_out)
        pltpu.sync_copy(ld_v, ld_out)


def _sc_gather(gamma, log_delta, idx):
    n = idx.shape[0]
    c = gamma.shape[1]
    fn = pl.kernel(
        _sc_gather_body,
        mesh=plsc.VectorSubcoreMesh(core_axis_name="c", subcore_axis_name="s"),
        out_type=[jax.ShapeDtypeStruct((n, c), jnp.float32),
                  jax.ShapeDtypeStruct((n, c), jnp.float32)],
        scratch_types=[pltpu.VMEM((n,), jnp.int32),
                       pltpu.VMEM((n, c), jnp.float32),
                       pltpu.VMEM((n, c), jnp.float32),
                       pltpu.SemaphoreType.DMA],
    )
    return fn(gamma, log_delta, idx)


def _apply_body(x_ref, g_ref, ld_ref, gt_ref, ldt_ref, m_ref, v_ref, o_ref):
    gb = g_ref[...]
    ldb = ld_ref[...]
    gt = gt_ref[...]
    ldt = ldt_ref[...]
    mean = m_ref[...]
    var = v_ref[...]

    sv = jnp.sqrt(var + 1e-8)
    r = jnp.sqrt(jnp.exp(ldt)) / jnp.sqrt(jnp.exp(ldb) + 1e-8)
    off = mean * (1.0 - r) + sv * (gt - gb * r)
    o_ref[...] = x_ref[...] * r + off


def _apply(x3, g_row, ld_row, g_tgt, ld_tgt, mean3, var3, rb):
    n, h, w = x3.shape
    c = mean3.shape[0]
    nper = c // rb
    grid = (n // rb,)
    row_spec = pl.BlockSpec((rb, 1, 1), lambda i: (i, 0, 0))
    per_spec = pl.BlockSpec((rb, 1, 1), lambda i: (i % nper, 0, 0))
    return pl.pallas_call(
        _apply_body,
        grid=grid,
        in_specs=[
            pl.BlockSpec((rb, h, w), lambda i: (i, 0, 0)),
            row_spec, row_spec, per_spec, per_spec, per_spec, per_spec,
        ],
        out_specs=pl.BlockSpec((rb, h, w), lambda i: (i, 0, 0)),
        out_shape=jax.ShapeDtypeStruct((n, h, w), jnp.float32),
        compiler_params=pltpu.CompilerParams(
            dimension_semantics=("parallel",)),
    )(x3, g_row, ld_row, g_tgt, ld_tgt, mean3, var3)


def kernel(x, batch, gamma, log_delta, running_mean, running_var, target_batch):
    b, c, h, w = x.shape
    hw = h * w

    tgt = jnp.full((8,), target_batch, dtype=jnp.int32)
    idx = jnp.concatenate([batch.astype(jnp.int32), tgt])

    cp = ((c + 127) // 128) * 128
    pad = ((0, 0), (0, cp - c))
    g_all, ld_all = _sc_gather(jnp.pad(gamma, pad), jnp.pad(log_delta, pad), idx)
    g_all = g_all[:, :c]
    ld_all = ld_all[:, :c]

    g_row = g_all[:b].reshape(b * c, 1, 1)
    ld_row = ld_all[:b].reshape(b * c, 1, 1)
    g_tgt = g_all[b].reshape(c, 1, 1)
    ld_tgt = ld_all[b].reshape(c, 1, 1)
    mean3 = running_mean.reshape(c, 1, 1)
    var3 = running_var.reshape(c, 1, 1)

    x3 = x.reshape(b * c, h, w)
    out = _apply(x3, g_row, ld_row, g_tgt, ld_tgt, mean3, var3, rb=48)
    return out.reshape(b, c, h, w)
